# Initial kernel scaffold; baseline (speedup 1.0000x reference)
import jax
import jax.numpy as jnp
from jax.experimental import pallas as pl


def kernel(x, edge_index, x0, weight, bias):
    raise NotImplementedError("write your pallas kernel here")


# speedup vs baseline: 4.8486x; 4.8486x over previous
import functools

import jax
import jax.numpy as jnp
from jax import lax
from jax.experimental import pallas as pl
from jax.experimental.pallas import tpu as pltpu
from jax.experimental.pallas import tpu_sc as plsc

N = 10000
D = 256
H = 128
E = 160000
NSUB = 16
EPT = E // NSUB
CH = 80
NCH = EPT // CH
RCH = 40
NRCH = N // RCH
ALPHA = 0.1
THETA = 0.5

MM_BLK = 2000


def _mm_body(x_ref, w_ref, o_ref):
    o_ref[0] = jnp.dot(x_ref[...], w_ref[...],
                       preferred_element_type=jnp.float32,
                       precision=jax.lax.Precision.HIGHEST)


def _matmul_split(x, w):
    return pl.pallas_call(
        _mm_body,
        grid=(N // MM_BLK, 2),
        in_specs=[
            pl.BlockSpec((MM_BLK, D), lambda i, c: (i, 0)),
            pl.BlockSpec((D, H), lambda i, c: (0, c)),
        ],
        out_specs=pl.BlockSpec((1, MM_BLK, H), lambda i, c: (c, i, 0)),
        out_shape=jax.ShapeDtypeStruct((2, N, H), jnp.float32),
    )(x, w)


def _combine_body(agg_ref, x0_ref, b_ref, o_ref):
    a = jnp.concatenate([agg_ref[0], agg_ref[1]], axis=1)
    o_ref[...] = a + (THETA * ALPHA) * x0_ref[...] + b_ref[...]


def _combine(agg2, x0, bias2d):
    return pl.pallas_call(
        _combine_body,
        grid=(N // MM_BLK,),
        in_specs=[
            pl.BlockSpec((2, MM_BLK, H), lambda i: (0, i, 0)),
            pl.BlockSpec((MM_BLK, D), lambda i: (i, 0)),
            pl.BlockSpec((1, D), lambda i: (0, 0)),
        ],
        out_specs=pl.BlockSpec((MM_BLK, D), lambda i: (i, 0)),
        out_shape=jax.ShapeDtypeStruct((N, D), jnp.float32),
    )(agg2, x0, bias2d)


def _sc_scatter(xw2, src_all, dst_all, zrows):
    mesh = plsc.VectorSubcoreMesh(core_axis_name="c", subcore_axis_name="s")

    @functools.partial(
        pl.kernel,
        mesh=mesh,
        out_type=jax.ShapeDtypeStruct((2 * N, H), jnp.float32),
        scratch_types=[
            pltpu.VMEM((NCH, CH), jnp.int32),
            pltpu.VMEM((NCH, CH), jnp.int32),
            pltpu.VMEM((CH, H), jnp.float32),
            pltpu.VMEM((RCH, H), jnp.float32),
            pltpu.VMEM_SHARED((N, H), jnp.float32),
            pltpu.SemaphoreType.DMA,
        ],
    )
    def k(xw_hbm, src_hbm, dst_hbm, z_hbm, out_hbm,
          src_v, dst_v, rows_v, obuf, agg_sh, sem):
        c = lax.axis_index("c")
        s = lax.axis_index("s")
        pltpu.sync_copy(src_hbm.at[c * NSUB + s], src_v)
        pltpu.sync_copy(dst_hbm.at[s], dst_v)
        pltpu.sync_copy(z_hbm, obuf)
        for i in range(pl.cdiv(NRCH, NSUB)):
            cid = s + i * NSUB

            @pl.when(cid < NRCH)
            def _zero():
                pltpu.sync_copy(obuf, agg_sh.at[pl.ds(cid * RCH, RCH)])

        plsc.subcore_barrier()

        def body(j, carry):
            pltpu.async_copy(xw_hbm.at[src_v.at[j]], rows_v, sem).wait()
            pltpu.sync_copy(rows_v, agg_sh.at[dst_v.at[j]], add=True)
            return carry

        lax.fori_loop(0, NCH, body, 0)
        plsc.subcore_barrier()
        for i in range(pl.cdiv(NRCH, NSUB)):
            cid = s + i * NSUB

            @pl.when(cid < NRCH)
            def _writeout():
                pltpu.sync_copy(agg_sh.at[pl.ds(cid * RCH, RCH)], obuf)
                pltpu.sync_copy(obuf, out_hbm.at[pl.ds(c * N + cid * RCH, RCH)])

    return k(xw2, src_all, dst_all, zrows)


def kernel(x, edge_index, x0, weight, bias):
    ei = edge_index.astype(jnp.int32)
    src = ei[0]
    dst = ei[1]
    src_all = jnp.stack([src, src + N]).reshape(2 * NSUB, NCH, CH)
    dst_all = dst.reshape(NSUB, NCH, CH)
    w_scaled = (THETA * (1.0 - ALPHA)) * weight
    xw = _matmul_split(x, w_scaled).reshape(2 * N, H)
    zrows = jnp.zeros((RCH, H), jnp.float32)
    agg2 = _sc_scatter(xw, src_all, dst_all, zrows).reshape(2, N, H)
    return _combine(agg2, x0, bias.reshape(1, D))
